# baseline (speedup 1.0000x reference)
import math
import functools

import jax
import jax.numpy as jnp
from jax import lax
from jax.experimental import pallas as pl
from jax.experimental.pallas import tpu as pltpu


def _select_body(lT_ref, g_ref, *, n_tok, n_exp, batch_k):
    l = lT_ref[...]
    v = lax.bitcast_convert_type(l, jnp.int32)

    def vbit(i, t):
        cand = jnp.bitwise_or(t, jnp.left_shift(jnp.int32(1), 30 - i))
        cnt = jnp.sum((v >= cand).astype(jnp.int32), axis=1, keepdims=True)
        return jnp.where(cnt >= batch_k, cand, t)

    T = lax.fori_loop(0, 31, vbit, jnp.zeros((n_exp, 1), jnp.int32))
    gt = v > T
    eq = v == T
    cnt_gt = jnp.sum(gt.astype(jnp.int32), axis=1, keepdims=True)
    need = batch_k - cnt_gt
    idx = lax.broadcasted_iota(jnp.int32, (n_exp, n_tok), 1)
    nbits_i = max(1, (2 * n_tok - 1).bit_length())

    def ibit(i, t):
        cand = jnp.bitwise_or(t, jnp.left_shift(jnp.int32(1), nbits_i - 1 - i))
        c = jnp.sum((eq & (idx < cand)).astype(jnp.int32),
                    axis=1, keepdims=True)
        return jnp.where(c <= need, cand, t)

    I = lax.fori_loop(0, nbits_i, ibit, jnp.zeros((n_exp, 1), jnp.int32))
    mask = gt | (eq & (idx < I))
    mx = jnp.max(jnp.where(mask, l, -jnp.inf), axis=1, keepdims=True)
    gu = jnp.where(mask, jnp.exp(l - mx), 0.0)
    den = jnp.sum(gu, axis=1, keepdims=True)
    gT = gu / den
    eye = (lax.broadcasted_iota(jnp.int32, (n_exp, n_exp), 0) ==
           lax.broadcasted_iota(jnp.int32, (n_exp, n_exp), 1)
           ).astype(jnp.float32)
    g_ref[...] = lax.dot_general(gT, eye, (((0,), (0,)), ((), ())),
                                 preferred_element_type=jnp.float32,
                                 precision=lax.Precision.HIGHEST)


def _expert_body(x_ref, ew_ref, eb_ref, g_ref, out_ref, oe_ref, temp_ref,
                 *, n_exp, out_dim):
    p = pl.program_id(1)
    blk = x_ref.shape[0]

    @pl.when(p < n_exp)
    def _compute():
        w = ew_ref[p]
        y = lax.dot_general(x_ref[...], w, (((1,), (1,)), ((), ())),
                            preferred_element_type=jnp.float32) + eb_ref[p]
        onehot = (lax.broadcasted_iota(jnp.int32, (blk, n_exp), 1) == p
                  ).astype(jnp.float32)
        gcol = jnp.sum(g_ref[...] * onehot, axis=1, keepdims=True)
        oe = y * gcol

        @pl.when(p == 0)
        def _():
            temp_ref[...] = oe

        @pl.when(p > 0)
        def _():
            temp_ref[...] += oe

        for s in range(n_exp):
            @pl.when(p == s)
            def _():
                oe_ref[:, s * out_dim:(s + 1) * out_dim] = oe

    for s in range(n_exp):
        @pl.when(p == n_exp + s)
        def _():
            out_ref[...] = (oe_ref[:, s * out_dim:(s + 1) * out_dim]
                            + temp_ref[...])


def kernel(x, rW1, rb1, rW1g, rb1g, rW2, rb2, eW, eb):
    n_tok, d_model = x.shape
    n_exp, out_dim, _ = eW.shape
    batch_k = math.ceil(2 / n_exp * n_tok)
    eblk = 512
    e_blocks = n_tok // eblk
    x_bf = x.astype(jnp.bfloat16)
    eW_bf = eW.astype(jnp.bfloat16)

    h1 = x @ rW1.T + rb1
    glu_mask = jax.nn.relu(x @ rW1g.T + rb1g)
    h1 = h1 * glu_mask
    h1 = jax.nn.relu(h1)
    logits = h1 @ rW2.T + rb2
    l = jax.nn.softmax(logits, axis=0)
    lT = l.T

    g = pl.pallas_call(
        functools.partial(_select_body, n_tok=n_tok, n_exp=n_exp,
                          batch_k=batch_k),
        in_specs=[pl.BlockSpec((n_exp, n_tok), lambda: (0, 0))],
        out_specs=pl.BlockSpec((n_tok, n_exp), lambda: (0, 0)),
        out_shape=jax.ShapeDtypeStruct((n_tok, n_exp), jnp.float32),
    )(lT)

    out = pl.pallas_call(
        functools.partial(_expert_body, n_exp=n_exp, out_dim=out_dim),
        grid=(e_blocks, 2 * n_exp),
        in_specs=[
            pl.BlockSpec((eblk, d_model), lambda tb, p: (tb, 0)),
            pl.BlockSpec((n_exp, out_dim, d_model), lambda tb, p: (0, 0, 0)),
            pl.BlockSpec((n_exp, 1, out_dim), lambda tb, p: (0, 0, 0)),
            pl.BlockSpec((eblk, n_exp), lambda tb, p: (tb, 0)),
        ],
        out_specs=pl.BlockSpec(
            (eblk, out_dim),
            lambda tb, p: (tb, jnp.maximum(p - n_exp, 0))),
        out_shape=jax.ShapeDtypeStruct((n_tok, n_exp * out_dim), jnp.float32),
        scratch_shapes=[pltpu.VMEM((eblk, n_exp * out_dim), jnp.float32),
                        pltpu.VMEM((eblk, out_dim), jnp.float32)],
        compiler_params=pltpu.CompilerParams(
            dimension_semantics=("parallel", "arbitrary"),
            vmem_limit_bytes=62 * 1024 * 1024),
    )(x_bf, eW_bf, eb.reshape(n_exp, 1, out_dim), g)
    return out


# speedup vs baseline: 1.0993x; 1.0993x over previous
import math
import functools

import jax
import jax.numpy as jnp
from jax import lax
from jax.experimental import pallas as pl
from jax.experimental.pallas import tpu as pltpu


def _select_body(lT_ref, g_ref, *, n_tok, n_exp, batch_k):
    l = lT_ref[...]
    v = lax.bitcast_convert_type(l, jnp.int32)

    def vbit(i, t):
        cand = jnp.bitwise_or(t, jnp.left_shift(jnp.int32(1), 30 - i))
        cnt = jnp.sum((v >= cand).astype(jnp.int32), axis=1, keepdims=True)
        return jnp.where(cnt >= batch_k, cand, t)

    T = lax.fori_loop(0, 31, vbit, jnp.zeros((n_exp, 1), jnp.int32))
    gt = v > T
    eq = v == T
    cnt_gt = jnp.sum(gt.astype(jnp.int32), axis=1, keepdims=True)
    need = batch_k - cnt_gt
    idx = lax.broadcasted_iota(jnp.int32, (n_exp, n_tok), 1)
    nbits_i = max(1, (2 * n_tok - 1).bit_length())

    def ibit(i, t):
        cand = jnp.bitwise_or(t, jnp.left_shift(jnp.int32(1), nbits_i - 1 - i))
        c = jnp.sum((eq & (idx < cand)).astype(jnp.int32),
                    axis=1, keepdims=True)
        return jnp.where(c <= need, cand, t)

    I = lax.fori_loop(0, nbits_i, ibit, jnp.zeros((n_exp, 1), jnp.int32))
    mask = gt | (eq & (idx < I))
    mx = jnp.max(jnp.where(mask, l, -jnp.inf), axis=1, keepdims=True)
    gu = jnp.where(mask, jnp.exp(l - mx), 0.0)
    den = jnp.sum(gu, axis=1, keepdims=True)
    gT = gu / den
    eye = (lax.broadcasted_iota(jnp.int32, (n_exp, n_exp), 0) ==
           lax.broadcasted_iota(jnp.int32, (n_exp, n_exp), 1)
           ).astype(jnp.float32)
    g_ref[...] = lax.dot_general(gT, eye, (((0,), (0,)), ((), ())),
                                 preferred_element_type=jnp.float32,
                                 precision=lax.Precision.HIGHEST)


def _expert_body(x_ref, ew_ref, eb_ref, g_ref, out_ref, temp_ref,
                 *, n_exp, out_dim):
    e = pl.program_id(1)
    blk = x_ref.shape[0]
    w = ew_ref[0]
    y = lax.dot_general(x_ref[...], w, (((1,), (1,)), ((), ())),
                        preferred_element_type=jnp.float32) + eb_ref[0]
    onehot = (lax.broadcasted_iota(jnp.int32, (blk, n_exp), 1) == e
              ).astype(jnp.float32)
    gcol = jnp.sum(g_ref[...] * onehot, axis=1, keepdims=True)
    oe = y * gcol

    @pl.when(e == 0)
    def _():
        temp_ref[...] = oe

    @pl.when(e > 0)
    def _():
        temp_ref[...] += oe

    for s in range(n_exp):
        @pl.when(e == s)
        def _():
            out_ref[:, s * out_dim:(s + 1) * out_dim] = oe

    @pl.when(e == n_exp - 1)
    def _():
        t = temp_ref[...]
        for s in range(n_exp):
            out_ref[:, s * out_dim:(s + 1) * out_dim] += t


def kernel(x, rW1, rb1, rW1g, rb1g, rW2, rb2, eW, eb):
    n_tok, d_model = x.shape
    n_exp, out_dim, _ = eW.shape
    batch_k = math.ceil(2 / n_exp * n_tok)
    eblk = 512
    e_blocks = n_tok // eblk
    x_bf = x.astype(jnp.bfloat16)
    eW_bf = eW.astype(jnp.bfloat16)

    h1 = x @ rW1.T + rb1
    glu_mask = jax.nn.relu(x @ rW1g.T + rb1g)
    h1 = h1 * glu_mask
    h1 = jax.nn.relu(h1)
    logits = h1 @ rW2.T + rb2
    l = jax.nn.softmax(logits, axis=0)
    lT = l.T

    g = pl.pallas_call(
        functools.partial(_select_body, n_tok=n_tok, n_exp=n_exp,
                          batch_k=batch_k),
        in_specs=[pl.BlockSpec((n_exp, n_tok), lambda: (0, 0))],
        out_specs=pl.BlockSpec((n_tok, n_exp), lambda: (0, 0)),
        out_shape=jax.ShapeDtypeStruct((n_tok, n_exp), jnp.float32),
    )(lT)

    out = pl.pallas_call(
        functools.partial(_expert_body, n_exp=n_exp, out_dim=out_dim),
        grid=(e_blocks, n_exp),
        in_specs=[
            pl.BlockSpec((eblk, d_model), lambda tb, e: (tb, 0)),
            pl.BlockSpec((1, out_dim, d_model), lambda tb, e: (e, 0, 0)),
            pl.BlockSpec((1, 1, out_dim), lambda tb, e: (e, 0, 0)),
            pl.BlockSpec((eblk, n_exp), lambda tb, e: (tb, 0)),
        ],
        out_specs=pl.BlockSpec((eblk, n_exp * out_dim), lambda tb, e: (tb, 0)),
        out_shape=jax.ShapeDtypeStruct((n_tok, n_exp * out_dim), jnp.float32),
        scratch_shapes=[pltpu.VMEM((eblk, out_dim), jnp.float32)],
        compiler_params=pltpu.CompilerParams(
            dimension_semantics=("parallel", "arbitrary"),
            vmem_limit_bytes=60 * 1024 * 1024),
    )(x_bf, eW_bf, eb.reshape(n_exp, 1, out_dim), g)
    return out


# speedup vs baseline: 1.1526x; 1.0485x over previous
import math
import functools

import jax
import jax.numpy as jnp
from jax import lax
from jax.experimental import pallas as pl
from jax.experimental.pallas import tpu as pltpu


def _select_body(lT_ref, g_ref, *, n_tok, n_exp, batch_k):
    l = lT_ref[...]
    v = lax.bitcast_convert_type(l, jnp.int32)

    def vbit(i, t):
        cand = jnp.bitwise_or(t, jnp.left_shift(jnp.int32(1), 30 - i))
        cnt = jnp.sum((v >= cand).astype(jnp.int32), axis=1, keepdims=True)
        return jnp.where(cnt >= batch_k, cand, t)

    T = lax.fori_loop(0, 31, vbit, jnp.zeros((n_exp, 1), jnp.int32))
    gt = v > T
    eq = v == T
    cnt_gt = jnp.sum(gt.astype(jnp.int32), axis=1, keepdims=True)
    need = batch_k - cnt_gt
    idx = lax.broadcasted_iota(jnp.int32, (n_exp, n_tok), 1)
    nbits_i = max(1, (2 * n_tok - 1).bit_length())

    def ibit(i, t):
        cand = jnp.bitwise_or(t, jnp.left_shift(jnp.int32(1), nbits_i - 1 - i))
        c = jnp.sum((eq & (idx < cand)).astype(jnp.int32),
                    axis=1, keepdims=True)
        return jnp.where(c <= need, cand, t)

    I = lax.fori_loop(0, nbits_i, ibit, jnp.zeros((n_exp, 1), jnp.int32))
    mask = gt | (eq & (idx < I))
    mx = jnp.max(jnp.where(mask, l, -jnp.inf), axis=1, keepdims=True)
    gu = jnp.where(mask, jnp.exp(l - mx), 0.0)
    den = jnp.sum(gu, axis=1, keepdims=True)
    gT = gu / den
    eye = (lax.broadcasted_iota(jnp.int32, (n_exp, n_exp), 0) ==
           lax.broadcasted_iota(jnp.int32, (n_exp, n_exp), 1)
           ).astype(jnp.float32)
    g_ref[...] = lax.dot_general(gT, eye, (((0,), (0,)), ((), ())),
                                 preferred_element_type=jnp.float32,
                                 precision=lax.Precision.HIGHEST)


def _expert_body(x_ref, ew_ref, eb_ref, g_ref, out_ref, temp_ref,
                 *, n_exp, out_dim):
    e = pl.program_id(1)
    blk = x_ref.shape[0]
    w = ew_ref[0]
    xb = x_ref[...].astype(jnp.bfloat16)
    y = lax.dot_general(xb, w, (((1,), (1,)), ((), ())),
                        preferred_element_type=jnp.float32) + eb_ref[0]
    onehot = (lax.broadcasted_iota(jnp.int32, (blk, n_exp), 1) == e
              ).astype(jnp.float32)
    gcol = jnp.sum(g_ref[...] * onehot, axis=1, keepdims=True)
    oe = y * gcol

    @pl.when(e == 0)
    def _():
        temp_ref[...] = oe

    @pl.when(e > 0)
    def _():
        temp_ref[...] += oe

    for s in range(n_exp):
        @pl.when(e == s)
        def _():
            out_ref[:, s * out_dim:(s + 1) * out_dim] = oe

    @pl.when(e == n_exp - 1)
    def _():
        t = temp_ref[...]
        for s in range(n_exp):
            out_ref[:, s * out_dim:(s + 1) * out_dim] += t


def kernel(x, rW1, rb1, rW1g, rb1g, rW2, rb2, eW, eb):
    n_tok, d_model = x.shape
    n_exp, out_dim, _ = eW.shape
    batch_k = math.ceil(2 / n_exp * n_tok)
    eblk = 512
    e_blocks = n_tok // eblk
    eW_bf = eW.astype(jnp.bfloat16)

    h1 = x @ rW1.T + rb1
    glu_mask = jax.nn.relu(x @ rW1g.T + rb1g)
    h1 = h1 * glu_mask
    h1 = jax.nn.relu(h1)
    logits = h1 @ rW2.T + rb2
    l = jax.nn.softmax(logits, axis=0)
    lT = l.T

    g = pl.pallas_call(
        functools.partial(_select_body, n_tok=n_tok, n_exp=n_exp,
                          batch_k=batch_k),
        in_specs=[pl.BlockSpec((n_exp, n_tok), lambda: (0, 0))],
        out_specs=pl.BlockSpec((n_tok, n_exp), lambda: (0, 0)),
        out_shape=jax.ShapeDtypeStruct((n_tok, n_exp), jnp.float32),
    )(lT)

    out = pl.pallas_call(
        functools.partial(_expert_body, n_exp=n_exp, out_dim=out_dim),
        grid=(e_blocks, n_exp),
        in_specs=[
            pl.BlockSpec((eblk, d_model), lambda tb, e: (tb, 0)),
            pl.BlockSpec((1, out_dim, d_model), lambda tb, e: (e, 0, 0)),
            pl.BlockSpec((1, 1, out_dim), lambda tb, e: (e, 0, 0)),
            pl.BlockSpec((eblk, n_exp), lambda tb, e: (tb, 0)),
        ],
        out_specs=pl.BlockSpec((eblk, n_exp * out_dim), lambda tb, e: (tb, 0)),
        out_shape=jax.ShapeDtypeStruct((n_tok, n_exp * out_dim), jnp.float32),
        scratch_shapes=[pltpu.VMEM((eblk, out_dim), jnp.float32)],
        compiler_params=pltpu.CompilerParams(
            dimension_semantics=("parallel", "arbitrary"),
            vmem_limit_bytes=60 * 1024 * 1024),
    )(x, eW_bf, eb.reshape(n_exp, 1, out_dim), g)
    return out
